# baseline (speedup 1.0000x reference)
import jax
import jax.numpy as jnp
from jax import lax
from jax.experimental import pallas as pl
from jax.experimental.pallas import tpu as pltpu

N = 256
FDIM = 128
HID = 16
T = 8


def _mm(a, b):
    return lax.dot_general(a, b, (((1,), (0,)), ((), ())),
                           preferred_element_type=jnp.float32)


def _mm_t(a, b):
    return lax.dot_general(a, b, (((0,), (0,)), ((), ())),
                           preferred_element_type=jnp.float32)


def _outer(a, b):
    return lax.dot_general(a, b, (((1,), (1,)), ((), ())),
                           preferred_element_type=jnp.float32)


def _gru_kernel(y_ref, w0_ref, w1_ref, w2_ref, bzr_ref,
                whhA_ref, whhB_ref, whhC_ref, bhh_ref,
                wred_ref, bred_ref, wm0_ref, bm0_ref, wm1_ref, bm1_ref,
                out_ref):
    row = lax.broadcasted_iota(jnp.int32, (N, N), 0)
    col = lax.broadcasted_iota(jnp.int32, (N, N), 1)
    offdiag = (row != col).astype(jnp.float32)
    ones_col = jnp.ones((N, 1), dtype=jnp.float32)

    w0 = w0_ref[...]
    w1 = w1_ref[...]
    w2 = w2_ref[...]
    bzr = bzr_ref[0]
    whhA = whhA_ref[...]
    whhB = whhB_ref[...]
    whhC = whhC_ref[...]
    bhh = bhh_ref[0]

    H = jnp.zeros((N, HID), dtype=jnp.float32)
    for t in range(T):
        A = y_ref[0, t, :, :N] * offdiag
        deg = _mm(A, ones_col)
        dinv = jnp.where(deg > 0,
                         lax.rsqrt(jnp.maximum(deg, 1e-12)),
                         0.0)
        S = A * _outer(-dinv, dinv)

        X = y_ref[0, t, :, N:]
        V0 = jnp.concatenate([X, H], axis=1)
        V1 = _mm_t(S, V0)
        V2 = _mm_t(S, V1)
        P = _mm(V0, w0) + _mm(V1, w1) + _mm(V2, w2) + bzr

        Z = jax.nn.sigmoid(P[:, :HID])
        R = jax.nn.sigmoid(P[:, HID:2 * HID])

        HR = H * R
        g1 = _mm_t(S, HR)
        g2 = _mm_t(S, g1)
        hpre = (P[:, 2 * HID:] + _mm(HR, whhA) + _mm(g1, whhB)
                + _mm(g2, whhC) + bhh)
        Htil = jnp.tanh(hpre)
        H = Z * H + (1.0 - Z) * Htil

    h = jax.nn.relu(_mm(H, wred_ref[...]) + bred_ref[0])
    o = _mm_t(h, wm0_ref[...]) + bm0_ref[...]
    o = _mm(o, wm1_ref[...]) + bm1_ref[...]
    out_ref[0] = o


@jax.jit
def kernel(y, Wxz, bxz, Whz, bhz, Wxr, bxr, Whr, bhr, Wxh, bxh, Whh, bhh,
           Wred, bred, Wm0, bm0, Wm1, bm1):
    B = y.shape[0]
    f32 = jnp.float32
    zh = jnp.zeros((HID, HID), f32)

    def stack(wx_list, wh_list):
        top = jnp.concatenate(wx_list, axis=1)
        bot = jnp.concatenate(wh_list, axis=1)
        return jnp.concatenate([top, bot], axis=0)

    w0 = stack([Wxz[0] - Wxz[2], Wxr[0] - Wxr[2], Wxh[0] - Wxh[2]],
               [Whz[0] - Whz[2], Whr[0] - Whr[2], zh])
    w1 = stack([Wxz[1], Wxr[1], Wxh[1]], [Whz[1], Whr[1], zh])
    w2 = stack([2.0 * Wxz[2], 2.0 * Wxr[2], 2.0 * Wxh[2]],
               [2.0 * Whz[2], 2.0 * Whr[2], zh])
    bzr = jnp.concatenate([bxz + bhz, bxr + bhr, bxh])[None, :]

    whhA = Whh[0] - Whh[2]
    whhB = Whh[1]
    whhC = 2.0 * Whh[2]
    bhh2 = bhh[None, :]
    bred2 = bred[None, :]
    bm02 = bm0[None, :]
    bm12 = bm1[None, :]

    full = lambda shape: pl.BlockSpec(shape, lambda b: (0,) * len(shape))
    out = pl.pallas_call(
        _gru_kernel,
        grid=(B,),
        in_specs=[
            pl.BlockSpec((1, T, N, N + FDIM), lambda b: (b, 0, 0, 0)),
            full((N // 2 + HID, 3 * HID)),
            full((N // 2 + HID, 3 * HID)),
            full((N // 2 + HID, 3 * HID)),
            full((1, 3 * HID)),
            full((HID, HID)),
            full((HID, HID)),
            full((HID, HID)),
            full((1, HID)),
            full((HID, 1)),
            full((1, 1)),
            full((N, 32)),
            full((1, 32)),
            full((32, HID)),
            full((1, HID)),
        ],
        out_specs=pl.BlockSpec((1, 1, HID), lambda b: (b, 0, 0)),
        out_shape=jax.ShapeDtypeStruct((B, 1, HID), jnp.float32),
        compiler_params=pltpu.CompilerParams(
            dimension_semantics=("parallel",)),
    )(y, w0, w1, w2, bzr, whhA, whhB, whhC, bhh2,
      Wred, bred2, Wm0, bm02, Wm1, bm12)
    return out.reshape(B, HID)


# speedup vs baseline: 1.0575x; 1.0575x over previous
import jax
import jax.numpy as jnp
from jax import lax
from jax.experimental import pallas as pl
from jax.experimental.pallas import tpu as pltpu

N = 256
FDIM = 128
HID = 16
T = 8
BPP = 2


def _mm(a, b):
    return lax.dot_general(a, b, (((1,), (0,)), ((), ())),
                           preferred_element_type=jnp.float32)


def _mm_t(a, b):
    return lax.dot_general(a, b, (((0,), (0,)), ((), ())),
                           preferred_element_type=jnp.float32)


def _outer(a, b):
    return lax.dot_general(a, b, (((1,), (1,)), ((), ())),
                           preferred_element_type=jnp.float32)


def _gru_kernel(y_ref, w0_ref, w1_ref, w2_ref, bzr_ref,
                whhA_ref, whhB_ref, whhC_ref, bhh_ref,
                wred_ref, bred_ref, wm0_ref, bm0_ref, wm1_ref, bm1_ref,
                out_ref):
    row = lax.broadcasted_iota(jnp.int32, (N, N), 0)
    col = lax.broadcasted_iota(jnp.int32, (N, N), 1)
    offdiag = (row != col).astype(jnp.float32)
    ones_col = jnp.ones((N, 1), dtype=jnp.float32)

    w0 = w0_ref[...]
    w1 = w1_ref[...]
    w2 = w2_ref[...]
    bzr = bzr_ref[0]
    whhA = whhA_ref[...]
    whhB = whhB_ref[...]
    whhC = whhC_ref[...]
    bhh = bhh_ref[0]

    Hs = [jnp.zeros((N, HID), dtype=jnp.float32) for _ in range(BPP)]
    for t in range(T):
        for i in range(BPP):
            H = Hs[i]
            A = y_ref[i, t, :, :N] * offdiag
            deg = _mm(A, ones_col)
            dinv = jnp.where(deg > 0,
                             lax.rsqrt(jnp.maximum(deg, 1e-12)),
                             0.0)
            S = A * _outer(-dinv, dinv)

            X = y_ref[i, t, :, N:]
            V0 = jnp.concatenate([X, H], axis=1)
            V1 = _mm_t(S, V0)
            V2 = _mm_t(S, V1)
            P = _mm(V0, w0) + _mm(V1, w1) + _mm(V2, w2) + bzr

            Z = jax.nn.sigmoid(P[:, :HID])
            R = jax.nn.sigmoid(P[:, HID:2 * HID])

            HR = H * R
            g1 = _mm_t(S, HR)
            g2 = _mm_t(S, g1)
            hpre = (P[:, 2 * HID:] + _mm(HR, whhA) + _mm(g1, whhB)
                    + _mm(g2, whhC) + bhh)
            Htil = jnp.tanh(hpre)
            Hs[i] = Z * H + (1.0 - Z) * Htil

    for i in range(BPP):
        h = jax.nn.relu(_mm(Hs[i], wred_ref[...]) + bred_ref[0])
        o = _mm_t(h, wm0_ref[...]) + bm0_ref[...]
        o = _mm(o, wm1_ref[...]) + bm1_ref[...]
        out_ref[i] = o


@jax.jit
def kernel(y, Wxz, bxz, Whz, bhz, Wxr, bxr, Whr, bhr, Wxh, bxh, Whh, bhh,
           Wred, bred, Wm0, bm0, Wm1, bm1):
    B = y.shape[0]
    f32 = jnp.float32
    zh = jnp.zeros((HID, HID), f32)

    def stack(wx_list, wh_list):
        top = jnp.concatenate(wx_list, axis=1)
        bot = jnp.concatenate(wh_list, axis=1)
        return jnp.concatenate([top, bot], axis=0)

    w0 = stack([Wxz[0] - Wxz[2], Wxr[0] - Wxr[2], Wxh[0] - Wxh[2]],
               [Whz[0] - Whz[2], Whr[0] - Whr[2], zh])
    w1 = stack([Wxz[1], Wxr[1], Wxh[1]], [Whz[1], Whr[1], zh])
    w2 = stack([2.0 * Wxz[2], 2.0 * Wxr[2], 2.0 * Wxh[2]],
               [2.0 * Whz[2], 2.0 * Whr[2], zh])
    bzr = jnp.concatenate([bxz + bhz, bxr + bhr, bxh])[None, :]

    whhA = Whh[0] - Whh[2]
    whhB = Whh[1]
    whhC = 2.0 * Whh[2]
    bhh2 = bhh[None, :]
    bred2 = bred[None, :]
    bm02 = bm0[None, :]
    bm12 = bm1[None, :]

    full = lambda shape: pl.BlockSpec(shape, lambda b: (0,) * len(shape))
    out = pl.pallas_call(
        _gru_kernel,
        grid=(B // BPP,),
        in_specs=[
            pl.BlockSpec((BPP, T, N, N + FDIM), lambda b: (b, 0, 0, 0)),
            full((N // 2 + HID, 3 * HID)),
            full((N // 2 + HID, 3 * HID)),
            full((N // 2 + HID, 3 * HID)),
            full((1, 3 * HID)),
            full((HID, HID)),
            full((HID, HID)),
            full((HID, HID)),
            full((1, HID)),
            full((HID, 1)),
            full((1, 1)),
            full((N, 32)),
            full((1, 32)),
            full((32, HID)),
            full((1, HID)),
        ],
        out_specs=pl.BlockSpec((BPP, 1, HID), lambda b: (b, 0, 0)),
        out_shape=jax.ShapeDtypeStruct((B, 1, HID), jnp.float32),
        compiler_params=pltpu.CompilerParams(
            dimension_semantics=("parallel",)),
    )(y, w0, w1, w2, bzr, whhA, whhB, whhC, bhh2,
      Wred, bred2, Wm0, bm02, Wm1, bm12)
    return out.reshape(B, HID)


# speedup vs baseline: 1.4364x; 1.3584x over previous
import jax
import jax.numpy as jnp
from jax import lax
from jax.experimental import pallas as pl
from jax.experimental.pallas import tpu as pltpu

N = 256
FDIM = 128
HID = 16
T = 8
BPP = 2


def _mm(a, b):
    return lax.dot_general(a, b, (((1,), (0,)), ((), ())),
                           preferred_element_type=jnp.float32)


def _mm_t(a, b):
    return lax.dot_general(a, b, (((0,), (0,)), ((), ())),
                           preferred_element_type=jnp.float32)


def _outer(a, b):
    return lax.dot_general(a, b, (((1,), (1,)), ((), ())),
                           preferred_element_type=jnp.float32)


def _gru_kernel(y_ref, w0_ref, w1_ref, w2_ref, bzr_ref,
                whhA_ref, whhB_ref, whhC_ref, bhh_ref,
                wred_ref, bred_ref, wm0_ref, bm0_ref, wm1_ref, bm1_ref,
                out_ref):
    row = lax.broadcasted_iota(jnp.int32, (N, N), 0)
    col = lax.broadcasted_iota(jnp.int32, (N, N), 1)
    offdiag = (row != col).astype(jnp.float32)
    ones_col = jnp.ones((N, 1), dtype=jnp.float32)

    w0 = w0_ref[...]
    w1 = w1_ref[...]
    w2 = w2_ref[...]
    bzr = bzr_ref[0]
    whhA = whhA_ref[...]
    whhB = whhB_ref[...]
    whhC = whhC_ref[...]
    bhh = bhh_ref[0]

    whh_cat = jnp.concatenate([whhA, whhB, whhC], axis=0)

    Hs = [jnp.zeros((N, HID), dtype=jnp.float32) for _ in range(BPP)]
    rng = range(BPP)
    for t in range(T):
        A = [y_ref[i, t, :, :N] * offdiag for i in rng]
        deg = [_mm(A[i], ones_col) for i in rng]
        dinv = [jnp.where(deg[i] > 0,
                          lax.rsqrt(jnp.maximum(deg[i], 1e-12)),
                          0.0) for i in rng]
        S = [A[i] * _outer(-dinv[i], dinv[i]) for i in rng]

        V0 = [jnp.concatenate([y_ref[i, t, :, N:], Hs[i]], axis=1)
              for i in rng]
        V1 = [_mm_t(S[i], V0[i]) for i in rng]
        V2 = [_mm_t(S[i], V1[i]) for i in rng]
        P = [_mm(V0[i], w0) + _mm(V1[i], w1) + _mm(V2[i], w2) + bzr
             for i in rng]

        Z = [jax.nn.sigmoid(P[i][:, :HID]) for i in rng]
        R = [jax.nn.sigmoid(P[i][:, HID:2 * HID]) for i in rng]

        HR = [Hs[i] * R[i] for i in rng]
        g1 = [_mm_t(S[i], HR[i]) for i in rng]
        g2 = [_mm_t(S[i], g1[i]) for i in rng]
        hcat = [jnp.concatenate([HR[i], g1[i], g2[i]], axis=1) for i in rng]
        hpre = [P[i][:, 2 * HID:] + _mm(hcat[i], whh_cat) + bhh for i in rng]
        Htil = [jnp.tanh(hpre[i]) for i in rng]
        Hs = [Z[i] * Hs[i] + (1.0 - Z[i]) * Htil[i] for i in rng]

    for i in range(BPP):
        h = jax.nn.relu(_mm(Hs[i], wred_ref[...]) + bred_ref[0])
        o = _mm_t(h, wm0_ref[...]) + bm0_ref[...]
        o = _mm(o, wm1_ref[...]) + bm1_ref[...]
        out_ref[i] = o


@jax.jit
def kernel(y, Wxz, bxz, Whz, bhz, Wxr, bxr, Whr, bhr, Wxh, bxh, Whh, bhh,
           Wred, bred, Wm0, bm0, Wm1, bm1):
    B = y.shape[0]
    f32 = jnp.float32
    zh = jnp.zeros((HID, HID), f32)

    def stack(wx_list, wh_list):
        top = jnp.concatenate(wx_list, axis=1)
        bot = jnp.concatenate(wh_list, axis=1)
        return jnp.concatenate([top, bot], axis=0)

    w0 = stack([Wxz[0] - Wxz[2], Wxr[0] - Wxr[2], Wxh[0] - Wxh[2]],
               [Whz[0] - Whz[2], Whr[0] - Whr[2], zh])
    w1 = stack([Wxz[1], Wxr[1], Wxh[1]], [Whz[1], Whr[1], zh])
    w2 = stack([2.0 * Wxz[2], 2.0 * Wxr[2], 2.0 * Wxh[2]],
               [2.0 * Whz[2], 2.0 * Whr[2], zh])
    bzr = jnp.concatenate([bxz + bhz, bxr + bhr, bxh])[None, :]

    whhA = Whh[0] - Whh[2]
    whhB = Whh[1]
    whhC = 2.0 * Whh[2]
    bhh2 = bhh[None, :]
    bred2 = bred[None, :]
    bm02 = bm0[None, :]
    bm12 = bm1[None, :]

    full = lambda shape: pl.BlockSpec(shape, lambda b: (0,) * len(shape))
    out = pl.pallas_call(
        _gru_kernel,
        grid=(B // BPP,),
        in_specs=[
            pl.BlockSpec((BPP, T, N, N + FDIM), lambda b: (b, 0, 0, 0)),
            full((N // 2 + HID, 3 * HID)),
            full((N // 2 + HID, 3 * HID)),
            full((N // 2 + HID, 3 * HID)),
            full((1, 3 * HID)),
            full((HID, HID)),
            full((HID, HID)),
            full((HID, HID)),
            full((1, HID)),
            full((HID, 1)),
            full((1, 1)),
            full((N, 32)),
            full((1, 32)),
            full((32, HID)),
            full((1, HID)),
        ],
        out_specs=pl.BlockSpec((BPP, 1, HID), lambda b: (b, 0, 0)),
        out_shape=jax.ShapeDtypeStruct((B, 1, HID), jnp.float32),
        compiler_params=pltpu.CompilerParams(
            dimension_semantics=("parallel",)),
    )(y, w0, w1, w2, bzr, whhA, whhB, whhC, bhh2,
      Wred, bred2, Wm0, bm02, Wm1, bm12)
    return out.reshape(B, HID)


# speedup vs baseline: 1.9551x; 1.3611x over previous
import jax
import jax.numpy as jnp
from jax import lax
from jax.experimental import pallas as pl
from jax.experimental.pallas import tpu as pltpu

N = 256
FDIM = 128
HID = 16
T = 8
BPP = 4


def _mm(a, b):
    return lax.dot_general(a, b, (((1,), (0,)), ((), ())),
                           preferred_element_type=jnp.float32)


def _mm_t(a, b):
    return lax.dot_general(a, b, (((0,), (0,)), ((), ())),
                           preferred_element_type=jnp.float32)


def _outer(a, b):
    return lax.dot_general(a, b, (((1,), (1,)), ((), ())),
                           preferred_element_type=jnp.float32)


def _gru_kernel(y_ref, w0_ref, w1_ref, w2_ref, bzr_ref,
                whhA_ref, whhB_ref, whhC_ref, bhh_ref,
                wred_ref, bred_ref, wm0_ref, bm0_ref, wm1_ref, bm1_ref,
                out_ref):
    row = lax.broadcasted_iota(jnp.int32, (N, N), 0)
    col = lax.broadcasted_iota(jnp.int32, (N, N), 1)
    offdiag = (row != col).astype(jnp.float32)
    ones_col = jnp.ones((N, 1), dtype=jnp.float32)

    w0 = w0_ref[...]
    w1 = w1_ref[...]
    w2 = w2_ref[...]
    bzr = bzr_ref[0]
    whhA = whhA_ref[...]
    whhB = whhB_ref[...]
    whhC = whhC_ref[...]
    bhh = bhh_ref[0]

    whh_cat = jnp.concatenate([whhA, whhB, whhC], axis=0)

    Hs = [jnp.zeros((N, HID), dtype=jnp.float32) for _ in range(BPP)]
    rng = range(BPP)
    for t in range(T):
        A = [y_ref[i, t, :, :N] * offdiag for i in rng]
        deg = [_mm(A[i], ones_col) for i in rng]
        dinv = [jnp.where(deg[i] > 0,
                          lax.rsqrt(jnp.maximum(deg[i], 1e-12)),
                          0.0) for i in rng]
        S = [A[i] * _outer(-dinv[i], dinv[i]) for i in rng]

        V0 = [jnp.concatenate([y_ref[i, t, :, N:], Hs[i]], axis=1)
              for i in rng]
        V1 = [_mm_t(S[i], V0[i]) for i in rng]
        V2 = [_mm_t(S[i], V1[i]) for i in rng]
        P = [_mm(V0[i], w0) + _mm(V1[i], w1) + _mm(V2[i], w2) + bzr
             for i in rng]

        Z = [jax.nn.sigmoid(P[i][:, :HID]) for i in rng]
        R = [jax.nn.sigmoid(P[i][:, HID:2 * HID]) for i in rng]

        HR = [Hs[i] * R[i] for i in rng]
        g1 = [_mm_t(S[i], HR[i]) for i in rng]
        g2 = [_mm_t(S[i], g1[i]) for i in rng]
        hcat = [jnp.concatenate([HR[i], g1[i], g2[i]], axis=1) for i in rng]
        hpre = [P[i][:, 2 * HID:] + _mm(hcat[i], whh_cat) + bhh for i in rng]
        Htil = [jnp.tanh(hpre[i]) for i in rng]
        Hs = [Z[i] * Hs[i] + (1.0 - Z[i]) * Htil[i] for i in rng]

    for i in range(BPP):
        h = jax.nn.relu(_mm(Hs[i], wred_ref[...]) + bred_ref[0])
        o = _mm_t(h, wm0_ref[...]) + bm0_ref[...]
        o = _mm(o, wm1_ref[...]) + bm1_ref[...]
        out_ref[i] = o


@jax.jit
def kernel(y, Wxz, bxz, Whz, bhz, Wxr, bxr, Whr, bhr, Wxh, bxh, Whh, bhh,
           Wred, bred, Wm0, bm0, Wm1, bm1):
    B = y.shape[0]
    f32 = jnp.float32
    zh = jnp.zeros((HID, HID), f32)

    def stack(wx_list, wh_list):
        top = jnp.concatenate(wx_list, axis=1)
        bot = jnp.concatenate(wh_list, axis=1)
        return jnp.concatenate([top, bot], axis=0)

    w0 = stack([Wxz[0] - Wxz[2], Wxr[0] - Wxr[2], Wxh[0] - Wxh[2]],
               [Whz[0] - Whz[2], Whr[0] - Whr[2], zh])
    w1 = stack([Wxz[1], Wxr[1], Wxh[1]], [Whz[1], Whr[1], zh])
    w2 = stack([2.0 * Wxz[2], 2.0 * Wxr[2], 2.0 * Wxh[2]],
               [2.0 * Whz[2], 2.0 * Whr[2], zh])
    bzr = jnp.concatenate([bxz + bhz, bxr + bhr, bxh])[None, :]

    whhA = Whh[0] - Whh[2]
    whhB = Whh[1]
    whhC = 2.0 * Whh[2]
    bhh2 = bhh[None, :]
    bred2 = bred[None, :]
    bm02 = bm0[None, :]
    bm12 = bm1[None, :]

    full = lambda shape: pl.BlockSpec(shape, lambda b: (0,) * len(shape))
    out = pl.pallas_call(
        _gru_kernel,
        grid=(B // BPP,),
        in_specs=[
            pl.BlockSpec((BPP, T, N, N + FDIM), lambda b: (b, 0, 0, 0)),
            full((N // 2 + HID, 3 * HID)),
            full((N // 2 + HID, 3 * HID)),
            full((N // 2 + HID, 3 * HID)),
            full((1, 3 * HID)),
            full((HID, HID)),
            full((HID, HID)),
            full((HID, HID)),
            full((1, HID)),
            full((HID, 1)),
            full((1, 1)),
            full((N, 32)),
            full((1, 32)),
            full((32, HID)),
            full((1, HID)),
        ],
        out_specs=pl.BlockSpec((BPP, 1, HID), lambda b: (b, 0, 0)),
        out_shape=jax.ShapeDtypeStruct((B, 1, HID), jnp.float32),
        compiler_params=pltpu.CompilerParams(
            dimension_semantics=("parallel",)),
    )(y, w0, w1, w2, bzr, whhA, whhB, whhC, bhh2,
      Wred, bred2, Wm0, bm02, Wm1, bm12)
    return out.reshape(B, HID)


# speedup vs baseline: 2.0082x; 1.0272x over previous
import jax
import jax.numpy as jnp
from jax import lax
from jax.experimental import pallas as pl
from jax.experimental.pallas import tpu as pltpu

N = 256
FDIM = 128
HID = 16
T = 8
BPP = 8


def _mm(a, b):
    return lax.dot_general(a, b, (((1,), (0,)), ((), ())),
                           preferred_element_type=jnp.float32)


def _mm_t(a, b):
    return lax.dot_general(a, b, (((0,), (0,)), ((), ())),
                           preferred_element_type=jnp.float32)


def _outer(a, b):
    return lax.dot_general(a, b, (((1,), (1,)), ((), ())),
                           preferred_element_type=jnp.float32)


def _gru_kernel(y_ref, w0_ref, w1_ref, w2_ref, bzr_ref,
                whhA_ref, whhB_ref, whhC_ref, bhh_ref,
                wred_ref, bred_ref, wm0_ref, bm0_ref, wm1_ref, bm1_ref,
                out_ref):
    row = lax.broadcasted_iota(jnp.int32, (N, N), 0)
    col = lax.broadcasted_iota(jnp.int32, (N, N), 1)
    offdiag = (row != col).astype(jnp.float32)
    ones_col = jnp.ones((N, 1), dtype=jnp.float32)

    w0 = w0_ref[...]
    w1 = w1_ref[...]
    w2 = w2_ref[...]
    bzr = bzr_ref[0]
    whhA = whhA_ref[...]
    whhB = whhB_ref[...]
    whhC = whhC_ref[...]
    bhh = bhh_ref[0]

    whh_cat = jnp.concatenate([whhA, whhB, whhC], axis=0)

    Hs = [jnp.zeros((N, HID), dtype=jnp.float32) for _ in range(BPP)]
    rng = range(BPP)
    for t in range(T):
        A = [y_ref[i, t, :, :N] * offdiag for i in rng]
        deg = [_mm(A[i], ones_col) for i in rng]
        dinv = [jnp.where(deg[i] > 0,
                          lax.rsqrt(jnp.maximum(deg[i], 1e-12)),
                          0.0) for i in rng]
        S = [A[i] * _outer(-dinv[i], dinv[i]) for i in rng]

        V0 = [jnp.concatenate([y_ref[i, t, :, N:], Hs[i]], axis=1)
              for i in rng]
        V1 = [_mm_t(S[i], V0[i]) for i in rng]
        V2 = [_mm_t(S[i], V1[i]) for i in rng]
        P = [_mm(V0[i], w0) + _mm(V1[i], w1) + _mm(V2[i], w2) + bzr
             for i in rng]

        Z = [jax.nn.sigmoid(P[i][:, :HID]) for i in rng]
        R = [jax.nn.sigmoid(P[i][:, HID:2 * HID]) for i in rng]

        HR = [Hs[i] * R[i] for i in rng]
        g1 = [_mm_t(S[i], HR[i]) for i in rng]
        g2 = [_mm_t(S[i], g1[i]) for i in rng]
        hcat = [jnp.concatenate([HR[i], g1[i], g2[i]], axis=1) for i in rng]
        hpre = [P[i][:, 2 * HID:] + _mm(hcat[i], whh_cat) + bhh for i in rng]
        Htil = [jnp.tanh(hpre[i]) for i in rng]
        Hs = [Z[i] * Hs[i] + (1.0 - Z[i]) * Htil[i] for i in rng]

    for i in range(BPP):
        h = jax.nn.relu(_mm(Hs[i], wred_ref[...]) + bred_ref[0])
        o = _mm_t(h, wm0_ref[...]) + bm0_ref[...]
        o = _mm(o, wm1_ref[...]) + bm1_ref[...]
        out_ref[i] = o


@jax.jit
def kernel(y, Wxz, bxz, Whz, bhz, Wxr, bxr, Whr, bhr, Wxh, bxh, Whh, bhh,
           Wred, bred, Wm0, bm0, Wm1, bm1):
    B = y.shape[0]
    f32 = jnp.float32
    zh = jnp.zeros((HID, HID), f32)

    def stack(wx_list, wh_list):
        top = jnp.concatenate(wx_list, axis=1)
        bot = jnp.concatenate(wh_list, axis=1)
        return jnp.concatenate([top, bot], axis=0)

    w0 = stack([Wxz[0] - Wxz[2], Wxr[0] - Wxr[2], Wxh[0] - Wxh[2]],
               [Whz[0] - Whz[2], Whr[0] - Whr[2], zh])
    w1 = stack([Wxz[1], Wxr[1], Wxh[1]], [Whz[1], Whr[1], zh])
    w2 = stack([2.0 * Wxz[2], 2.0 * Wxr[2], 2.0 * Wxh[2]],
               [2.0 * Whz[2], 2.0 * Whr[2], zh])
    bzr = jnp.concatenate([bxz + bhz, bxr + bhr, bxh])[None, :]

    whhA = Whh[0] - Whh[2]
    whhB = Whh[1]
    whhC = 2.0 * Whh[2]
    bhh2 = bhh[None, :]
    bred2 = bred[None, :]
    bm02 = bm0[None, :]
    bm12 = bm1[None, :]

    full = lambda shape: pl.BlockSpec(shape, lambda b: (0,) * len(shape))
    out = pl.pallas_call(
        _gru_kernel,
        grid=(B // BPP,),
        in_specs=[
            pl.BlockSpec((BPP, T, N, N + FDIM), lambda b: (b, 0, 0, 0)),
            full((N // 2 + HID, 3 * HID)),
            full((N // 2 + HID, 3 * HID)),
            full((N // 2 + HID, 3 * HID)),
            full((1, 3 * HID)),
            full((HID, HID)),
            full((HID, HID)),
            full((HID, HID)),
            full((1, HID)),
            full((HID, 1)),
            full((1, 1)),
            full((N, 32)),
            full((1, 32)),
            full((32, HID)),
            full((1, HID)),
        ],
        out_specs=pl.BlockSpec((BPP, 1, HID), lambda b: (b, 0, 0)),
        out_shape=jax.ShapeDtypeStruct((B, 1, HID), jnp.float32),
        compiler_params=pltpu.CompilerParams(
            dimension_semantics=("parallel",)),
    )(y, w0, w1, w2, bzr, whhA, whhB, whhC, bhh2,
      Wred, bred2, Wm0, bm02, Wm1, bm12)
    return out.reshape(B, HID)
